# baseline (speedup 1.0000x reference)
import functools

import jax
import jax.numpy as jnp
from jax import lax
from jax.experimental import pallas as pl
from jax.experimental.pallas import tpu as pltpu
from jax.experimental.pallas import tpu_sc as plsc

_BATCH = 16384
_BITS = 128

_ROT_A = (13, 15, 26, 6)
_ROT_B = (17, 29, 16, 24)


def _threefry_bits(x1):
    ks = (jnp.uint32(0), jnp.uint32(42), jnp.uint32(0 ^ 42 ^ 0x1BD11BDA))

    def rotl(v, d):
        return (v << jnp.uint32(d)) | (v >> jnp.uint32(32 - d))

    x0 = x1
    x1 = x0 ^ rotl(x1, _ROT_A[0])
    for r in _ROT_A[1:]:
        x0 = x0 + x1
        x1 = rotl(x1, r)
        x1 = x0 ^ x1
    x0 = x0 + ks[1]
    x1 = x1 + (ks[2] + jnp.uint32(1))
    for i in range(1, 5):
        for r in (_ROT_A if i % 2 == 0 else _ROT_B):
            x0 = x0 + x1
            x1 = rotl(x1, r)
            x1 = x0 ^ x1
        x0 = x0 + ks[(i + 1) % 3]
        x1 = x1 + (ks[(i + 2) % 3] + jnp.uint32(i + 1))
    return x0 ^ x1


def _u_from_bits(bits):
    return jax.lax.bitcast_convert_type(
        (bits >> jnp.uint32(9)) | jnp.uint32(0x3F800000), jnp.float32) - 1.0


_TC_ROWS_PER_BLOCK = 1024


def _tc_body(x_ref, o_ref):
    p = pl.program_id(0)
    shape = (_TC_ROWS_PER_BLOCK, _BITS)
    base = (p * _TC_ROWS_PER_BLOCK * _BITS + 42).astype(jnp.uint32)
    x1 = base + (
        jax.lax.broadcasted_iota(jnp.uint32, shape, 0) * jnp.uint32(_BITS)
        + jax.lax.broadcasted_iota(jnp.uint32, shape, 1))
    u = _u_from_bits(_threefry_bits(x1))
    o_ref[...] = jnp.where(u < x_ref[...], 1.0, 0.0)


def _tc_sample(x2, rows):
    return pl.pallas_call(
        _tc_body,
        grid=(rows // _TC_ROWS_PER_BLOCK,),
        in_specs=[pl.BlockSpec((_TC_ROWS_PER_BLOCK, 1), lambda p: (p, 0))],
        out_specs=pl.BlockSpec((_TC_ROWS_PER_BLOCK, _BITS), lambda p: (p, 0)),
        out_shape=jax.ShapeDtypeStruct((rows, _BITS), jnp.float32),
    )(x2)


_SC_WORKERS = 32


def _sc_body(row0, sc_rows, x_hbm, out_hbm, x_v, out_v):
    rows_per_w = sc_rows // _SC_WORKERS
    wid = lax.axis_index("s") * 2 + lax.axis_index("c")
    wbase = wid * rows_per_w
    pltpu.sync_copy(x_hbm.at[pl.ds(row0 + wbase, rows_per_w)], x_v)
    lane = lax.iota(jnp.int32, 16)

    def group_body(g, carry):
        xs = x_v[pl.ds(g * 16, 16)]
        xb = [jnp.broadcast_to(xs[j], (16,)) for j in range(16)]
        gbase = (row0 + wbase + g * 16) * _BITS + 42

        def col_body(c, carry2):
            cbase = gbase + c * 16
            for j in range(16):
                ctr = jnp.full((16,), cbase + j * _BITS, jnp.int32) + lane
                u = _u_from_bits(_threefry_bits(ctr.astype(jnp.uint32)))
                out_v[g * 16 + j, pl.ds(c * 16, 16)] = \
                    jnp.where(u < xb[j], 1.0, 0.0)
            return carry2

        lax.fori_loop(0, _BITS // 16, col_body, 0)
        return carry

    lax.fori_loop(0, rows_per_w // 16, group_body, 0)
    pltpu.sync_copy(out_v, out_hbm.at[pl.ds(wbase, rows_per_w)])


def _sc_sample(x, row0, sc_rows):
    rows_per_w = sc_rows // _SC_WORKERS
    mesh = plsc.VectorSubcoreMesh(core_axis_name="c", subcore_axis_name="s")
    f = pl.kernel(
        functools.partial(_sc_body, row0, sc_rows),
        out_type=jax.ShapeDtypeStruct((sc_rows, _BITS), jnp.float32),
        mesh=mesh,
        scratch_types=[
            pltpu.VMEM((rows_per_w,), jnp.float32),
            pltpu.VMEM((rows_per_w, _BITS), jnp.float32),
        ],
    )
    return f(x)


_SC_ROWS = 16384


def kernel(x):
    tc_rows = _BATCH - _SC_ROWS
    sc_out = _sc_sample(x, tc_rows, _SC_ROWS)
    if tc_rows == 0:
        return sc_out
    tc_out = _tc_sample(x[:tc_rows].reshape(tc_rows, 1), tc_rows)
    return jnp.concatenate([tc_out, sc_out], axis=0)


# speedup vs baseline: 2.2052x; 2.2052x over previous
import functools

import jax
import jax.numpy as jnp
from jax import lax
from jax.experimental import pallas as pl
from jax.experimental.pallas import tpu as pltpu
from jax.experimental.pallas import tpu_sc as plsc

_BATCH = 16384
_BITS = 128

_ROT_A = (13, 15, 26, 6)
_ROT_B = (17, 29, 16, 24)


def _threefry_bits(x1):
    ks = (jnp.uint32(0), jnp.uint32(42), jnp.uint32(0 ^ 42 ^ 0x1BD11BDA))

    def rotl(v, d):
        return (v << jnp.uint32(d)) | (v >> jnp.uint32(32 - d))

    x0 = x1
    x1 = x0 ^ rotl(x1, _ROT_A[0])
    for r in _ROT_A[1:]:
        x0 = x0 + x1
        x1 = rotl(x1, r)
        x1 = x0 ^ x1
    x0 = x0 + ks[1]
    x1 = x1 + (ks[2] + jnp.uint32(1))
    for i in range(1, 5):
        for r in (_ROT_A if i % 2 == 0 else _ROT_B):
            x0 = x0 + x1
            x1 = rotl(x1, r)
            x1 = x0 ^ x1
        x0 = x0 + ks[(i + 1) % 3]
        x1 = x1 + (ks[(i + 2) % 3] + jnp.uint32(i + 1))
    return x0 ^ x1


def _u_from_bits(bits):
    return jax.lax.bitcast_convert_type(
        (bits >> jnp.uint32(9)) | jnp.uint32(0x3F800000), jnp.float32) - 1.0


_TC_ROWS_PER_BLOCK = 1024


def _tc_body(x_ref, o_ref):
    p = pl.program_id(0)
    shape = (_TC_ROWS_PER_BLOCK, _BITS)
    base = (p * _TC_ROWS_PER_BLOCK * _BITS + 42).astype(jnp.uint32)
    x1 = base + (
        jax.lax.broadcasted_iota(jnp.uint32, shape, 0) * jnp.uint32(_BITS)
        + jax.lax.broadcasted_iota(jnp.uint32, shape, 1))
    u = _u_from_bits(_threefry_bits(x1))
    o_ref[...] = jnp.where(u < x_ref[...], 1.0, 0.0)


def _tc_sample(x2, rows):
    return pl.pallas_call(
        _tc_body,
        grid=(rows // _TC_ROWS_PER_BLOCK,),
        in_specs=[pl.BlockSpec((_TC_ROWS_PER_BLOCK, 1), lambda p: (p, 0))],
        out_specs=pl.BlockSpec((_TC_ROWS_PER_BLOCK, _BITS), lambda p: (p, 0)),
        out_shape=jax.ShapeDtypeStruct((rows, _BITS), jnp.float32),
    )(x2)


_SC_WORKERS = 32


def _sc_body(row0, sc_rows, x_hbm, out_hbm, x_v, out_v):
    rows_per_w = sc_rows // _SC_WORKERS
    wid = lax.axis_index("s") * 2 + lax.axis_index("c")
    wbase = wid * rows_per_w
    pltpu.sync_copy(x_hbm.at[pl.ds(row0 + wbase, rows_per_w)], x_v)
    lane = lax.iota(jnp.int32, 16)

    def group_body(g, carry):
        xs = x_v[pl.ds(g * 16, 16)]
        xb = [jnp.broadcast_to(xs[j], (16,)) for j in range(16)]
        gbase = (row0 + wbase + g * 16) * _BITS + 42

        def col_body(c, carry2):
            cbase = gbase + c * 16
            for j in range(16):
                ctr = jnp.full((16,), cbase + j * _BITS, jnp.int32) + lane
                u = _u_from_bits(_threefry_bits(ctr.astype(jnp.uint32)))
                out_v[g * 16 + j, pl.ds(c * 16, 16)] = \
                    jnp.where(u < xb[j], 1.0, 0.0)
            return carry2

        lax.fori_loop(0, _BITS // 16, col_body, 0)
        return carry

    lax.fori_loop(0, rows_per_w // 16, group_body, 0)
    pltpu.sync_copy(out_v, out_hbm.at[pl.ds(wbase, rows_per_w)])


def _sc_sample(x, row0, sc_rows):
    rows_per_w = sc_rows // _SC_WORKERS
    mesh = plsc.VectorSubcoreMesh(core_axis_name="c", subcore_axis_name="s")
    f = pl.kernel(
        functools.partial(_sc_body, row0, sc_rows),
        out_type=jax.ShapeDtypeStruct((sc_rows, _BITS), jnp.float32),
        mesh=mesh,
        scratch_types=[
            pltpu.VMEM((rows_per_w,), jnp.float32),
            pltpu.VMEM((rows_per_w, _BITS), jnp.float32),
        ],
    )
    return f(x)


_SC_ROWS = 4096


def kernel(x):
    tc_rows = _BATCH - _SC_ROWS
    sc_out = _sc_sample(x, tc_rows, _SC_ROWS)
    if tc_rows == 0:
        return sc_out
    tc_out = _tc_sample(x[:tc_rows].reshape(tc_rows, 1), tc_rows)
    return jnp.concatenate([tc_out, sc_out], axis=0)
